# baseline (device time: 121807 ns/iter reference)
import jax
import jax.numpy as jnp
from jax import lax
from jax.experimental import pallas as pl
from jax.experimental.pallas import tpu as pltpu

N_DEV = 8
HEADS_PER = 8
SQ = 256
SKV = 4096
DH = 128
DM = HEADS_PER * DH
BLK = 64
SCALE = 0.08838834764831843


def kernel(x, Wq, K_ext, V_ext, Wo):
    my = lax.axis_index("i")

    xb = x[0].astype(jnp.bfloat16)
    Wqb = Wq.astype(jnp.bfloat16)
    Wob = Wo.astype(jnp.bfloat16)
    Ks = lax.dynamic_slice(
        K_ext, (0, 0, my * HEADS_PER, 0), (1, SKV, HEADS_PER, DH))[0]
    Vs = lax.dynamic_slice(
        V_ext, (0, 0, my * HEADS_PER, 0), (1, SKV, HEADS_PER, DH))[0]
    Kb = jnp.transpose(Ks, (1, 0, 2)).astype(jnp.bfloat16)
    Vb = jnp.transpose(Vs, (1, 0, 2)).astype(jnp.bfloat16)

    def body(x_ref, wq_ref, k_ref, v_ref, wo_ref, out_ref,
             comm_ref, send_sems, recv_sems):
        my_pos = lax.axis_index("i")
        left = lax.rem(my_pos - 1 + N_DEV, N_DEV)
        right = lax.rem(my_pos + 1, N_DEV)

        barrier_sem = pltpu.get_barrier_semaphore()
        for nbr in (left, right):
            pl.semaphore_signal(
                barrier_sem, inc=1,
                device_id=(nbr,), device_id_type=pl.DeviceIdType.MESH,
            )
        pl.semaphore_wait(barrier_sem, 2)

        q = jnp.dot(x_ref[...], wq_ref[...],
                    preferred_element_type=jnp.float32)
        qb = q.astype(jnp.bfloat16)

        rowb = lax.broadcasted_iota(jnp.int32, (SQ, SKV), 0) // BLK
        colb = lax.broadcasted_iota(jnp.int32, (SQ, SKV), 1) // BLK
        mask = (rowb == colb) | (colb == 0) | (lax.rem(rowb + colb, 3) == 0)

        acc = jnp.zeros((SQ, DM), jnp.float32)
        for h in range(HEADS_PER):
            qh = qb[:, h * DH:(h + 1) * DH]
            s = lax.dot_general(
                qh, k_ref[h], (((1,), (1,)), ((), ())),
                preferred_element_type=jnp.float32) * SCALE
            s = jnp.where(mask, s, -1e9)
            m = jnp.max(s, axis=-1, keepdims=True)
            w = jnp.exp(s - m)
            w = w / jnp.sum(w, axis=-1, keepdims=True)
            ctx = jnp.dot(w.astype(jnp.bfloat16), v_ref[h],
                          preferred_element_type=jnp.float32)
            acc = acc + jnp.dot(
                ctx.astype(jnp.bfloat16), wo_ref[h * DH:(h + 1) * DH, :],
                preferred_element_type=jnp.float32)

        comm_ref[0] = acc.astype(jnp.bfloat16)
        for h in range(N_DEV - 1):
            rdma = pltpu.make_async_remote_copy(
                src_ref=comm_ref.at[h],
                dst_ref=comm_ref.at[h + 1],
                send_sem=send_sems.at[h],
                recv_sem=recv_sems.at[h],
                device_id=(right,),
                device_id_type=pl.DeviceIdType.MESH,
            )
            rdma.start()
            rdma.wait()

        total = acc
        for j in range(1, N_DEV):
            total = total + comm_ref[j][...].astype(jnp.float32)
        out_ref[0] = total

    out = pl.pallas_call(
        body,
        out_shape=jax.ShapeDtypeStruct((1, SQ, DM), jnp.float32),
        in_specs=[pl.BlockSpec(memory_space=pltpu.VMEM)] * 5,
        out_specs=pl.BlockSpec(memory_space=pltpu.VMEM),
        scratch_shapes=[
            pltpu.VMEM((N_DEV, SQ, DM), jnp.bfloat16),
            pltpu.SemaphoreType.DMA((N_DEV - 1,)),
            pltpu.SemaphoreType.DMA((N_DEV - 1,)),
        ],
        compiler_params=pltpu.CompilerParams(collective_id=0),
    )(xb, Wqb, Kb, Vb, Wob)
    return out


# device time: 92279 ns/iter; 1.3200x vs baseline; 1.3200x over previous
import jax
import jax.numpy as jnp
from jax import lax
from jax.experimental import pallas as pl
from jax.experimental.pallas import tpu as pltpu

N_DEV = 8
HEADS_PER = 8
SQ = 256
SKV = 4096
DH = 128
DM = HEADS_PER * DH
BLK = 64
SCALE = 0.08838834764831843


def kernel(x, Wq, K_ext, V_ext, Wo):
    my = lax.axis_index("i")

    xb = x[0].astype(jnp.bfloat16)
    Wqb = Wq.astype(jnp.bfloat16)
    Wob = Wo.astype(jnp.bfloat16)
    Ks = lax.dynamic_slice(
        K_ext, (0, 0, my * HEADS_PER, 0), (1, SKV, HEADS_PER, DH))[0]
    Vs = lax.dynamic_slice(
        V_ext, (0, 0, my * HEADS_PER, 0), (1, SKV, HEADS_PER, DH))[0]
    Kb = jnp.transpose(Ks, (1, 0, 2)).astype(jnp.bfloat16)
    Vb = jnp.transpose(Vs, (1, 0, 2)).astype(jnp.bfloat16)

    XOR_STAGES = (1, 3, 4)

    def body(x_ref, wq_ref, k_ref, v_ref, wo_ref, out_ref,
             send_ref, recv_ref, send_sems, recv_sems):
        my_pos = lax.axis_index("i")
        partners = [jnp.bitwise_xor(my_pos, c) for c in XOR_STAGES]

        barrier_sem = pltpu.get_barrier_semaphore()
        for nbr in partners:
            pl.semaphore_signal(
                barrier_sem, inc=1,
                device_id=(nbr,), device_id_type=pl.DeviceIdType.MESH,
            )
        pl.semaphore_wait(barrier_sem, len(partners))

        q = jnp.dot(x_ref[...], wq_ref[...],
                    preferred_element_type=jnp.float32)
        qb = q.astype(jnp.bfloat16)

        rowb = lax.broadcasted_iota(jnp.int32, (SQ, SKV), 0) // BLK
        colb = lax.broadcasted_iota(jnp.int32, (SQ, SKV), 1) // BLK
        mask = (rowb == colb) | (colb == 0) | (lax.rem(rowb + colb, 3) == 0)

        acc = jnp.zeros((SQ, DM), jnp.float32)
        for h in range(HEADS_PER):
            qh = qb[:, h * DH:(h + 1) * DH]
            s = lax.dot_general(
                qh, k_ref[h], (((1,), (1,)), ((), ())),
                preferred_element_type=jnp.float32) * SCALE
            s = jnp.where(mask, s, -1e9)
            m = jnp.max(s, axis=-1, keepdims=True)
            w = jnp.exp(s - m)
            w = w / jnp.sum(w, axis=-1, keepdims=True)
            ctx = jnp.dot(w.astype(jnp.bfloat16), v_ref[h],
                          preferred_element_type=jnp.float32)
            acc = acc + jnp.dot(
                ctx.astype(jnp.bfloat16), wo_ref[h * DH:(h + 1) * DH, :],
                preferred_element_type=jnp.float32)

        for s, partner in enumerate(partners):
            send_ref[s] = acc.astype(jnp.bfloat16)
            rdma = pltpu.make_async_remote_copy(
                src_ref=send_ref.at[s],
                dst_ref=recv_ref.at[s],
                send_sem=send_sems.at[s],
                recv_sem=recv_sems.at[s],
                device_id=(partner,),
                device_id_type=pl.DeviceIdType.MESH,
            )
            rdma.start()
            rdma.wait()
            acc = acc + recv_ref[s][...].astype(jnp.float32)

        out_ref[0] = acc

    out = pl.pallas_call(
        body,
        out_shape=jax.ShapeDtypeStruct((1, SQ, DM), jnp.float32),
        in_specs=[pl.BlockSpec(memory_space=pltpu.VMEM)] * 5,
        out_specs=pl.BlockSpec(memory_space=pltpu.VMEM),
        scratch_shapes=[
            pltpu.VMEM((3, SQ, DM), jnp.bfloat16),
            pltpu.VMEM((3, SQ, DM), jnp.bfloat16),
            pltpu.SemaphoreType.DMA((3,)),
            pltpu.SemaphoreType.DMA((3,)),
        ],
        compiler_params=pltpu.CompilerParams(collective_id=0),
    )(xb, Wqb, Kb, Vb, Wob)
    return out


# device time: 57358 ns/iter; 2.1236x vs baseline; 1.6088x over previous
import jax
import jax.numpy as jnp
from jax import lax
from jax.experimental import pallas as pl
from jax.experimental.pallas import tpu as pltpu

N_DEV = 8
HEADS_PER = 8
SQ = 256
SKV = 4096
DH = 128
DM = HEADS_PER * DH
BLK = 64
SCALE = 0.08838834764831843


def kernel(x, Wq, K_ext, V_ext, Wo):

    XOR_STAGES = (1, 3, 4)

    def body(x_ref, wq_ref, k_any, v_any, wo_ref, out_ref,
             kv_bufs, kv_sems, send_ref, recv_ref, send_sems, recv_sems):
        my_pos = lax.axis_index("i")
        partners = [jnp.bitwise_xor(my_pos, c) for c in XOR_STAGES]

        barrier_sem = pltpu.get_barrier_semaphore()
        for nbr in partners:
            pl.semaphore_signal(
                barrier_sem, inc=1,
                device_id=(nbr,), device_id_type=pl.DeviceIdType.MESH,
            )
        pl.semaphore_wait(barrier_sem, len(partners))

        def start_kv(h, slot):
            head = my_pos * HEADS_PER + h
            kcp = pltpu.make_async_copy(
                k_any.at[0, :, head, :], kv_bufs.at[slot, 0],
                kv_sems.at[slot, 0])
            vcp = pltpu.make_async_copy(
                v_any.at[0, :, head, :], kv_bufs.at[slot, 1],
                kv_sems.at[slot, 1])
            kcp.start()
            vcp.start()
            return kcp, vcp

        pending = start_kv(0, 0)

        xb = x_ref[0].astype(jnp.bfloat16)
        wqb = wq_ref[...].astype(jnp.bfloat16)
        wob = wo_ref[...].astype(jnp.bfloat16)
        q = jnp.dot(xb, wqb, preferred_element_type=jnp.float32)
        qb = q.astype(jnp.bfloat16)

        rowb = lax.broadcasted_iota(jnp.int32, (SQ, SKV), 0) // BLK
        colb = lax.broadcasted_iota(jnp.int32, (SQ, SKV), 1) // BLK
        mask = (rowb == colb) | (colb == 0) | (lax.rem(rowb + colb, 3) == 0)

        acc = jnp.zeros((SQ, DM), jnp.float32)
        for h in range(HEADS_PER):
            slot = h % 2
            pending[0].wait()
            pending[1].wait()
            if h + 1 < HEADS_PER:
                pending = start_kv(h + 1, (h + 1) % 2)
            kh = kv_bufs[slot, 0].astype(jnp.bfloat16)
            vh = kv_bufs[slot, 1].astype(jnp.bfloat16)
            qh = qb[:, h * DH:(h + 1) * DH]
            s = lax.dot_general(
                qh, kh, (((1,), (1,)), ((), ())),
                preferred_element_type=jnp.float32) * SCALE
            s = jnp.where(mask, s, -1e9)
            m = jnp.max(s, axis=-1, keepdims=True)
            w = jnp.exp(s - m)
            w = w / jnp.sum(w, axis=-1, keepdims=True)
            ctx = jnp.dot(w.astype(jnp.bfloat16), vh,
                          preferred_element_type=jnp.float32)
            acc = acc + jnp.dot(
                ctx.astype(jnp.bfloat16), wob[h * DH:(h + 1) * DH, :],
                preferred_element_type=jnp.float32)

        for s, partner in enumerate(partners):
            send_ref[s] = acc.astype(jnp.bfloat16)
            rdma = pltpu.make_async_remote_copy(
                src_ref=send_ref.at[s],
                dst_ref=recv_ref.at[s],
                send_sem=send_sems.at[s],
                recv_sem=recv_sems.at[s],
                device_id=(partner,),
                device_id_type=pl.DeviceIdType.MESH,
            )
            rdma.start()
            rdma.wait()
            acc = acc + recv_ref[s][...].astype(jnp.float32)

        out_ref[0] = acc

    out = pl.pallas_call(
        body,
        out_shape=jax.ShapeDtypeStruct((1, SQ, DM), jnp.float32),
        in_specs=[
            pl.BlockSpec(memory_space=pltpu.VMEM),
            pl.BlockSpec(memory_space=pltpu.VMEM),
            pl.BlockSpec(memory_space=pl.ANY),
            pl.BlockSpec(memory_space=pl.ANY),
            pl.BlockSpec(memory_space=pltpu.VMEM),
        ],
        out_specs=pl.BlockSpec(memory_space=pltpu.VMEM),
        scratch_shapes=[
            pltpu.VMEM((2, 2, SKV, DH), jnp.float32),
            pltpu.SemaphoreType.DMA((2, 2)),
            pltpu.VMEM((3, SQ, DM), jnp.bfloat16),
            pltpu.VMEM((3, SQ, DM), jnp.bfloat16),
            pltpu.SemaphoreType.DMA((3,)),
            pltpu.SemaphoreType.DMA((3,)),
        ],
        compiler_params=pltpu.CompilerParams(collective_id=0),
    )(x, Wq, K_ext, V_ext, Wo)
    return out
